# baseline (speedup 1.0000x reference)
import jax
import jax.numpy as jnp
from jax import lax
from jax.experimental import pallas as pl
from jax.experimental.pallas import tpu as pltpu
from jax.experimental.pallas import tpu_sc as plsc

N_NODES = 10000
N_EDGES = 320000
DIM = 128
NEG = 0.3

NC = 2
NS = 16
NW = NC * NS
E_PER_W = N_EDGES // NW
K = 40
K2 = 2 * K
CHUNKS = E_PER_W // K
N_PAD = 10112
CPAD = 2 * N_PAD
ROWS_PER_TILE = N_PAD // NS
CNT_PER_TILE = CPAD // NS
ZROWS = 128


def _mm_body(x_ref, w_ref, b_ref, t_out):
    i = pl.program_id(0)
    x = x_ref[...]
    w = w_ref[...]
    w1 = w[:, :DIM]
    w2 = w[:, DIM:]
    dn = (((1,), (1,)), ((), ()))
    a = lax.dot_general(x, w1 - w2, dn,
                        preferred_element_type=jnp.float32) + b_ref[...]
    bv = lax.dot_general(x, w2, dn, preferred_element_type=jnp.float32)
    t_out[...] = jnp.where(i < 25, a, bv)


def _node_matmuls(feature, W, b2d):
    rows = 400
    return pl.pallas_call(
        _mm_body,
        grid=(50,),
        in_specs=[
            pl.BlockSpec((rows, DIM), lambda i: (i % 25, 0)),
            pl.BlockSpec((DIM, 2 * DIM), lambda i: (0, 0)),
            pl.BlockSpec((1, DIM), lambda i: (0, 0)),
        ],
        out_specs=pl.BlockSpec((rows, DIM), lambda i: (i, 0)),
        out_shape=jax.ShapeDtypeStruct((2 * N_NODES, DIM), jnp.float32),
    )(feature, W, b2d)


def _sc_body(tab_hbm, gidx_hbm, dst_hbm, out_hbm, cnt_hbm,
             gix0, dix0, gix1, dix1, gix2, dix2, gix3, dix3,
             gbuf0, gbuf1, mbuf0, mbuf1, zbuf, ones, zc,
             accum, cnt_sp,
             ix0, ix1, ix2, ix3, ga0, ga1, ss0, ss1, cs0, cs1):
    c = lax.axis_index("c")
    s = lax.axis_index("s")
    wid = c * NS + s
    zero16 = jnp.zeros((16,), jnp.float32)
    one16 = jnp.ones((16,), jnp.float32)

    def zrow(r, _):
        for cc in range(DIM // 16):
            zbuf[r, pl.ds(cc * 16, 16)] = zero16
        return _
    lax.fori_loop(0, ZROWS, zrow, None)

    def fill1(r, _):
        ones[pl.ds(r * 16, 16)] = one16
        return _
    lax.fori_loop(0, K2 // 16, fill1, None)

    def fill0(r, _):
        zc[pl.ds(r * 16, 16)] = zero16
        return _
    lax.fori_loop(0, 640 // 16, fill0, None)

    rbase = s * ROWS_PER_TILE
    for j in range(4):
        pltpu.sync_copy(zbuf, accum.at[pl.ds(rbase + j * ZROWS, ZROWS)])
    pltpu.sync_copy(zbuf.at[pl.ds(0, ROWS_PER_TILE - 4 * ZROWS)],
                    accum.at[pl.ds(rbase + 4 * ZROWS, ROWS_PER_TILE - 4 * ZROWS)])
    cbase = s * CNT_PER_TILE
    pltpu.sync_copy(zc, cnt_sp.at[pl.ds(cbase, 640)])
    pltpu.sync_copy(zc.at[pl.ds(0, CNT_PER_TILE - 640)],
                    cnt_sp.at[pl.ds(cbase + 640, CNT_PER_TILE - 640)])
    plsc.subcore_barrier()

    isets = ((gix0, dix0, ix0), (gix1, dix1, ix1),
             (gix2, dix2, ix2), (gix3, dix3, ix3))
    gsets = ((gbuf0, ga0), (gbuf1, ga1))
    msets = ((mbuf0, ss0, cs0), (mbuf1, ss1, cs1))

    def start_idx(i, islot):
        gv, dv, sem = isets[islot]
        pltpu.async_copy(gidx_hbm.at[pl.ds((wid * CHUNKS + i) * K2, K2)], gv, sem)
        pltpu.async_copy(dst_hbm.at[pl.ds(wid * E_PER_W + i * K, K)], dv, sem)

    def wait_idx(i, islot):
        gv, dv, sem = isets[islot]
        pltpu.make_async_copy(gidx_hbm.at[pl.ds((wid * CHUNKS + i) * K2, K2)],
                              gv, sem).wait()
        pltpu.make_async_copy(dst_hbm.at[pl.ds(wid * E_PER_W + i * K, K)],
                              dv, sem).wait()

    def start_gather(islot, cur):
        gb, ga = gsets[cur]
        pltpu.async_copy(tab_hbm.at[isets[islot][0]], gb, ga)

    def wait_gather(islot, cur):
        gb, ga = gsets[cur]
        pltpu.make_async_copy(tab_hbm.at[isets[islot][0]], gb, ga).wait()

    def start_scatter(islot, mcur):
        mb, ssem, csem = msets[mcur]
        pltpu.async_copy(mb, accum.at[isets[islot][1]], ssem, add=True)
        pltpu.async_copy(ones, cnt_sp.at[isets[islot][0]], csem, add=True)

    def wait_scatter(islot, mcur):
        mb, ssem, csem = msets[mcur]
        pltpu.make_async_copy(mb, accum.at[isets[islot][1]], ssem).wait()
        pltpu.make_async_copy(ones, cnt_sp.at[isets[islot][0]], csem).wait()

    def compute(cur, mcur):
        gb = gsets[cur][0]
        mb = msets[mcur][0]

        @plsc.parallel_loop(0, K, 1, unroll=4)
        def _row(r):
            for cc in range(DIM // 16):
                sl = pl.ds(cc * 16, 16)
                m = gb[r, sl] + gb[K + r, sl]
                mb[r, sl] = jnp.where(m >= 0.0, m, NEG * m)

    def step(i, sub):
        cur = sub % 2

        @pl.when(i < CHUNKS)
        def _():
            @pl.when(i + 1 < CHUNKS)
            def _():
                wait_idx(i + 1, (sub + 1) % 4)
                start_gather((sub + 1) % 4, 1 - cur)
            wait_gather(sub, cur)

            @pl.when(i >= 2)
            def _():
                wait_scatter((sub + 2) % 4, cur)
            compute(cur, cur)
            start_scatter(sub, cur)

            @pl.when(i + 2 < CHUNKS)
            def _():
                start_idx(i + 2, (sub + 2) % 4)

    start_idx(0, 0)
    wait_idx(0, 0)
    start_gather(0, 0)
    start_idx(1, 1)

    def group(g, _):
        i0 = 4 * g
        step(i0, 0)
        step(i0 + 1, 1)
        step(i0 + 2, 2)
        step(i0 + 3, 3)
        return _
    lax.fori_loop(0, (CHUNKS + 3) // 4, group, None)

    wait_scatter((CHUNKS - 2) % 4, (CHUNKS - 2) % 2)
    wait_scatter((CHUNKS - 1) % 4, (CHUNKS - 1) % 2)
    plsc.subcore_barrier()

    pltpu.sync_copy(accum.at[pl.ds(rbase, ROWS_PER_TILE)],
                    out_hbm.at[c, pl.ds(rbase, ROWS_PER_TILE)])
    pltpu.sync_copy(cnt_sp.at[pl.ds(rbase, ROWS_PER_TILE)],
                    zc.at[pl.ds(0, ROWS_PER_TILE)])
    pltpu.sync_copy(zc.at[pl.ds(0, ROWS_PER_TILE)],
                    cnt_hbm.at[pl.ds(c * N_PAD + rbase, ROWS_PER_TILE)])


def _sc_aggregate(TAB, gidx, dst):
    mesh = plsc.VectorSubcoreMesh(core_axis_name="c", subcore_axis_name="s",
                                  num_cores=NC, num_subcores=NS)
    f = pl.kernel(
        _sc_body,
        out_type=[
            jax.ShapeDtypeStruct((NC, N_PAD, DIM), jnp.float32),
            jax.ShapeDtypeStruct((NC * N_PAD,), jnp.float32),
        ],
        mesh=mesh,
        scratch_types=(
            [pltpu.VMEM((K2,), jnp.int32), pltpu.VMEM((K,), jnp.int32)] * 4
            + [pltpu.VMEM((K2, DIM), jnp.float32)] * 2
            + [pltpu.VMEM((K, DIM), jnp.float32)] * 2
            + [pltpu.VMEM((ZROWS, DIM), jnp.float32),
               pltpu.VMEM((K2,), jnp.float32),
               pltpu.VMEM((640,), jnp.float32),
               pltpu.VMEM_SHARED((N_PAD, DIM), jnp.float32),
               pltpu.VMEM_SHARED((CPAD,), jnp.float32)]
            + [pltpu.SemaphoreType.DMA] * 10
        ),
    )
    return f(TAB, gidx, dst)


def _fin_body(p_ref, c_ref, o_ref):
    p = p_ref[...]
    t = p[0] + p[1]
    c = c_ref[...]
    cnt = c[0] + c[1]
    o_ref[...] = t / jnp.maximum(cnt, 1.0)


def _finalize(partial, counts3):
    grid = 79
    rows = N_PAD // grid
    return pl.pallas_call(
        _fin_body,
        grid=(grid,),
        in_specs=[
            pl.BlockSpec((NC, rows, DIM), lambda i: (0, i, 0)),
            pl.BlockSpec((NC, rows, 1), lambda i: (0, i, 0)),
        ],
        out_specs=pl.BlockSpec((rows, DIM), lambda i: (i, 0)),
        out_shape=jax.ShapeDtypeStruct((N_PAD, DIM), jnp.float32),
    )(partial, counts3)


def kernel(feature, edge_index, W, b):
    src = edge_index[0]
    dst = edge_index[1]
    gidx = jnp.concatenate(
        [dst.reshape(-1, K), src.reshape(-1, K) + N_NODES], axis=1).reshape(-1)
    TAB = _node_matmuls(feature, W, b.reshape(1, DIM))
    partial, counts = _sc_aggregate(TAB, gidx, dst)
    out = _finalize(partial, counts.reshape(NC, N_PAD, 1))
    return out[:N_NODES]


# speedup vs baseline: 1.1360x; 1.1360x over previous
import jax
import jax.numpy as jnp
from jax import lax
from jax.experimental import pallas as pl
from jax.experimental.pallas import tpu as pltpu
from jax.experimental.pallas import tpu_sc as plsc

N_NODES = 10000
N_EDGES = 320000
DIM = 128
NEG = 0.3

NC = 2
NS = 16
NW = NC * NS
E_PER_W = N_EDGES // NW
K = 40
CHUNKS = E_PER_W // K
N_PAD = 10112
ROWS_PER_TILE = N_PAD // NS
ZROWS = 128


def _mm_body(x_ref, w_ref, b_ref, a_out, b_out):
    x = x_ref[...]
    w = w_ref[...]
    w1 = w[:, :DIM]
    w2 = w[:, DIM:]
    dn = (((1,), (1,)), ((), ()))
    a_out[...] = lax.dot_general(x, w1 - w2, dn,
                                 preferred_element_type=jnp.float32) + b_ref[...]
    b_out[...] = lax.dot_general(x, w2, dn,
                                 preferred_element_type=jnp.float32)


def _node_matmuls(feature, W, b2d):
    grid = 25
    rows = N_NODES // grid
    return pl.pallas_call(
        _mm_body,
        grid=(grid,),
        in_specs=[
            pl.BlockSpec((rows, DIM), lambda i: (i, 0)),
            pl.BlockSpec((DIM, 2 * DIM), lambda i: (0, 0)),
            pl.BlockSpec((1, DIM), lambda i: (0, 0)),
        ],
        out_specs=[
            pl.BlockSpec((rows, DIM), lambda i: (i, 0)),
            pl.BlockSpec((rows, DIM), lambda i: (i, 0)),
        ],
        out_shape=[
            jax.ShapeDtypeStruct((N_NODES, DIM), jnp.float32),
            jax.ShapeDtypeStruct((N_NODES, DIM), jnp.float32),
        ],
    )(feature, W, b2d)


def _sc_body(a_hbm, b_hbm, src_hbm, dst_hbm, out_hbm, cnt_hbm,
             sidx0, didx0, sidx1, didx1, sidx2, didx2, sidx3, didx3,
             abuf, bbuf, abuf1, bbuf1, mbuf, mbuf1,
             zbuf, ones, zc, accum, cnt_sp,
             ga0, gb0, ga1, gb1, ix0, ix1, ix2, ix3, ss0, ss1, cs0, cs1):
    c = lax.axis_index("c")
    s = lax.axis_index("s")
    ebase = (c * NS + s) * E_PER_W
    zero16 = jnp.zeros((16,), jnp.float32)
    one16 = jnp.ones((16,), jnp.float32)

    def zrow(r, _):
        for cc in range(DIM // 16):
            zbuf[r, pl.ds(cc * 16, 16)] = zero16
        return _
    lax.fori_loop(0, ZROWS, zrow, None)

    def fill1(r, _):
        ones[pl.ds(r * 16, 16)] = one16
        zc[pl.ds(r * 16, 16)] = zero16
        return _
    lax.fori_loop(0, 640 // 16, fill1, None)

    rbase = s * ROWS_PER_TILE
    for j in range(4):
        pltpu.sync_copy(zbuf, accum.at[pl.ds(rbase + j * ZROWS, ZROWS)])
    pltpu.sync_copy(zbuf.at[pl.ds(0, ROWS_PER_TILE - 4 * ZROWS)],
                    accum.at[pl.ds(rbase + 4 * ZROWS, ROWS_PER_TILE - 4 * ZROWS)])
    pltpu.sync_copy(zc.at[pl.ds(0, ROWS_PER_TILE)],
                    cnt_sp.at[pl.ds(rbase, ROWS_PER_TILE)])
    plsc.subcore_barrier()

    isets = ((sidx0, didx0, ix0), (sidx1, didx1, ix1),
             (sidx2, didx2, ix2), (sidx3, didx3, ix3))
    gsets = ((abuf, bbuf, ga0, gb0), (abuf1, bbuf1, ga1, gb1))
    msets = ((mbuf, ss0, cs0), (mbuf1, ss1, cs1))

    def start_idx(i, islot):
        sv, dv, sem = isets[islot]
        pltpu.async_copy(src_hbm.at[pl.ds(ebase + i * K, K)], sv, sem)
        pltpu.async_copy(dst_hbm.at[pl.ds(ebase + i * K, K)], dv, sem)

    def wait_idx(i, islot):
        sv, dv, sem = isets[islot]
        pltpu.make_async_copy(src_hbm.at[pl.ds(ebase + i * K, K)], sv, sem).wait()
        pltpu.make_async_copy(dst_hbm.at[pl.ds(ebase + i * K, K)], dv, sem).wait()

    def start_gather(islot, cur):
        ab, bb, ga, gb = gsets[cur]
        pltpu.async_copy(a_hbm.at[isets[islot][1]], ab, ga)
        pltpu.async_copy(b_hbm.at[isets[islot][0]], bb, gb)

    def wait_gather(islot, cur):
        ab, bb, ga, gb = gsets[cur]
        pltpu.make_async_copy(a_hbm.at[isets[islot][1]], ab, ga).wait()
        pltpu.make_async_copy(b_hbm.at[isets[islot][0]], bb, gb).wait()

    def start_scatter(islot, mcur):
        mb, ssem, csem = msets[mcur]
        dv = isets[islot][1]
        pltpu.async_copy(mb, accum.at[dv], ssem, add=True)
        pltpu.async_copy(ones.at[pl.ds(0, K)], cnt_sp.at[dv], csem, add=True)

    def wait_scatter(islot, mcur):
        mb, ssem, csem = msets[mcur]
        dv = isets[islot][1]
        pltpu.make_async_copy(mb, accum.at[dv], ssem).wait()
        pltpu.make_async_copy(ones.at[pl.ds(0, K)], cnt_sp.at[dv], csem).wait()

    def compute(cur, mcur):
        ab, bb, _, _ = gsets[cur]
        mb = msets[mcur][0]

        @plsc.parallel_loop(0, K, 1, unroll=8)
        def _row(r):
            for cc in range(DIM // 16):
                sl = pl.ds(cc * 16, 16)
                m = ab[r, sl] + bb[r, sl]
                mb[r, sl] = jnp.where(m >= 0.0, m, NEG * m)

    def step(i, sub):
        cur = sub % 2

        @pl.when(i < CHUNKS)
        def _():
            @pl.when(i + 1 < CHUNKS)
            def _():
                wait_idx(i + 1, (sub + 1) % 4)
                start_gather((sub + 1) % 4, 1 - cur)
            wait_gather(sub, cur)

            @pl.when(i >= 2)
            def _():
                wait_scatter((sub + 2) % 4, cur)
            compute(cur, cur)
            start_scatter(sub, cur)

            @pl.when(i + 2 < CHUNKS)
            def _():
                start_idx(i + 2, (sub + 2) % 4)

    start_idx(0, 0)
    wait_idx(0, 0)
    start_gather(0, 0)
    start_idx(1, 1)

    def group(g, _):
        i0 = 4 * g
        step(i0, 0)
        step(i0 + 1, 1)
        step(i0 + 2, 2)
        step(i0 + 3, 3)
        return _
    lax.fori_loop(0, (CHUNKS + 3) // 4, group, None)

    wait_scatter((CHUNKS - 2) % 4, (CHUNKS - 2) % 2)
    wait_scatter((CHUNKS - 1) % 4, (CHUNKS - 1) % 2)
    plsc.subcore_barrier()

    pltpu.sync_copy(accum.at[pl.ds(rbase, ROWS_PER_TILE)],
                    out_hbm.at[c, pl.ds(rbase, ROWS_PER_TILE)])
    pltpu.sync_copy(cnt_sp.at[pl.ds(rbase, ROWS_PER_TILE)],
                    zc.at[pl.ds(0, ROWS_PER_TILE)])
    pltpu.sync_copy(zc.at[pl.ds(0, ROWS_PER_TILE)],
                    cnt_hbm.at[pl.ds(c * N_PAD + rbase, ROWS_PER_TILE)])


def _sc_aggregate(A, B, src, dst):
    mesh = plsc.VectorSubcoreMesh(core_axis_name="c", subcore_axis_name="s",
                                  num_cores=NC, num_subcores=NS)
    f = pl.kernel(
        _sc_body,
        out_type=[
            jax.ShapeDtypeStruct((NC, N_PAD, DIM), jnp.float32),
            jax.ShapeDtypeStruct((NC * N_PAD,), jnp.float32),
        ],
        mesh=mesh,
        scratch_types=(
            [pltpu.VMEM((K,), jnp.int32)] * 8
            + [pltpu.VMEM((K, DIM), jnp.float32)] * 6
            + [pltpu.VMEM((ZROWS, DIM), jnp.float32),
               pltpu.VMEM((640,), jnp.float32),
               pltpu.VMEM((640,), jnp.float32),
               pltpu.VMEM_SHARED((N_PAD, DIM), jnp.float32),
               pltpu.VMEM_SHARED((N_PAD,), jnp.float32)]
            + [pltpu.SemaphoreType.DMA] * 12
        ),
    )
    return f(A, B, src, dst)


def _fin_body(p_ref, c_ref, o_ref):
    p = p_ref[...]
    t = p[0] + p[1]
    c = c_ref[...]
    cnt = c[0] + c[1]
    o_ref[...] = t / jnp.maximum(cnt, 1.0)


def _finalize(partial, counts3):
    grid = 79
    rows = N_PAD // grid
    return pl.pallas_call(
        _fin_body,
        grid=(grid,),
        in_specs=[
            pl.BlockSpec((NC, rows, DIM), lambda i: (0, i, 0)),
            pl.BlockSpec((NC, rows, 1), lambda i: (0, i, 0)),
        ],
        out_specs=pl.BlockSpec((rows, DIM), lambda i: (i, 0)),
        out_shape=jax.ShapeDtypeStruct((N_PAD, DIM), jnp.float32),
    )(partial, counts3)


def kernel(feature, edge_index, W, b):
    src = edge_index[0]
    dst = edge_index[1]
    A, B = _node_matmuls(feature, W, b.reshape(1, DIM))
    partial, counts = _sc_aggregate(A, B, src, dst)
    out = _finalize(partial, counts.reshape(NC, N_PAD, 1))
    return out[:N_NODES]
